# baseline (speedup 1.0000x reference)
import functools

import jax
import jax.numpy as jnp
from jax import lax
from jax.experimental import pallas as pl
from jax.experimental.pallas import tpu as pltpu
from jax.experimental.pallas import tpu_sc as plsc

N = 10000
E = 320000
D = 128
NC = 2
NS = 16
NW = NC * NS
K = 128
EPAD = 327680
EPT = EPAD // NW
NCHUNK = EPT // K
NA = N + 8
NZT = 10
ZR = N // NZT

_MESH = plsc.VectorSubcoreMesh(core_axis_name="c", subcore_axis_name="s",
                               num_cores=NC, num_subcores=NS)


def _sc_cnt_body(dst_hbm, z_hbm, ones_hbm,
                 out_cnt, cnt_acc, dst_v, ones_v):
    c = lax.axis_index("c")
    s = lax.axis_index("s")
    wid = s * NC + c
    stripe = pl.ds(s * ZR, ZR)

    @pl.when(s < NZT)
    def _zero():
        pltpu.sync_copy(z_hbm, cnt_acc.at[stripe])

    pltpu.sync_copy(ones_hbm, ones_v)
    pltpu.sync_copy(dst_hbm.at[wid], dst_v)
    plsc.subcore_barrier()

    def body(j, carry):
        pltpu.sync_copy(ones_v, cnt_acc.at[dst_v.at[j]], add=True)
        return carry

    lax.fori_loop(0, NCHUNK, body, 0)
    plsc.subcore_barrier()

    @pl.when(s < NZT)
    def _drain():
        pltpu.sync_copy(cnt_acc.at[stripe], out_cnt.at[c, stripe])


def _sc_body(h_hbm, sd_hbm, z_hbm,
             out_rows, acc, sd0, sd1, rows0, rows1, sem0, sem1):
    c = lax.axis_index("c")
    s = lax.axis_index("s")
    wid = s * NC + c
    stripe = pl.ds(s * ZR, ZR)

    @pl.when(s < NZT)
    def _zero():
        pltpu.sync_copy(z_hbm, acc.at[stripe])

    plsc.subcore_barrier()

    pltpu.sync_copy(sd_hbm.at[wid, 0], sd0)
    pltpu.async_copy(h_hbm.at[sd0.at[0]], rows0, sem0)

    def pair(p, carry):
        j1 = 2 * p + 1
        pltpu.sync_copy(sd_hbm.at[wid, j1], sd1)
        pltpu.async_copy(h_hbm.at[sd1.at[0]], rows1, sem1)
        pltpu.make_async_copy(h_hbm.at[sd0.at[0]], rows0, sem0).wait()
        pltpu.sync_copy(rows0, acc.at[sd0.at[1]], add=True)

        @pl.when(j1 + 1 < NCHUNK)
        def _next():
            pltpu.sync_copy(sd_hbm.at[wid, j1 + 1], sd0)
            pltpu.async_copy(h_hbm.at[sd0.at[0]], rows0, sem0)

        pltpu.make_async_copy(h_hbm.at[sd1.at[0]], rows1, sem1).wait()
        pltpu.sync_copy(rows1, acc.at[sd1.at[1]], add=True)
        return carry

    lax.fori_loop(0, NCHUNK // 2, pair, 0)
    plsc.subcore_barrier()

    @pl.when(s < NZT)
    def _drain():
        pltpu.sync_copy(acc.at[stripe], out_rows.at[c, stripe])


_sc_cnt = pl.kernel(
    _sc_cnt_body,
    out_type=jax.ShapeDtypeStruct((NC, N, D), jnp.float32),
    mesh=_MESH,
    scratch_types=[
        pltpu.VMEM_SHARED((NA, D), jnp.float32),
        pltpu.VMEM((NCHUNK, K), jnp.int32),
        pltpu.VMEM((K, D), jnp.float32),
    ],
)

_sc_agg = pl.kernel(
    _sc_body,
    out_type=jax.ShapeDtypeStruct((NC, N, D), jnp.float32),
    mesh=_MESH,
    scratch_types=[
        pltpu.VMEM_SHARED((NA, D), jnp.float32),
        pltpu.VMEM((2, K), jnp.int32),
        pltpu.VMEM((2, K), jnp.int32),
        pltpu.VMEM((K, D), jnp.float32),
        pltpu.VMEM((K, D), jnp.float32),
        pltpu.SemaphoreType.DMA,
        pltpu.SemaphoreType.DMA,
    ],
)


BN = 1000


def _tc_body(p0, p1, c0, c1, h, wl, wr, b, o, *, act):
    cnt = c0[:, 0:1] + c1[:, 0:1]
    inv = 1.0 / jnp.maximum(cnt, 1.0)
    agg = (p0[:, :] + p1[:, :]) * inv
    y = (jnp.dot(agg, wl[:, :], preferred_element_type=jnp.float32)
         + jnp.dot(h[:, :], wr[:, :], preferred_element_type=jnp.float32)
         + b[:, :])
    o[:, :] = jnp.tanh(y) if act else y


def _tc_layer(parts, cnts, h, Wl, Wr, b, act):
    return pl.pallas_call(
        functools.partial(_tc_body, act=act),
        grid=(N // BN,),
        in_specs=[
            pl.BlockSpec((BN, D), lambda i: (i, 0)),
            pl.BlockSpec((BN, D), lambda i: (i, 0)),
            pl.BlockSpec((BN, D), lambda i: (i, 0)),
            pl.BlockSpec((BN, D), lambda i: (i, 0)),
            pl.BlockSpec((BN, D), lambda i: (i, 0)),
            pl.BlockSpec((D, D), lambda i: (0, 0)),
            pl.BlockSpec((D, D), lambda i: (0, 0)),
            pl.BlockSpec((1, D), lambda i: (0, 0)),
        ],
        out_specs=pl.BlockSpec((BN, D), lambda i: (i, 0)),
        out_shape=jax.ShapeDtypeStruct((N, D), jnp.float32),
    )(parts[0], parts[1], cnts[0], cnts[1], h, Wl, Wr, b.reshape(1, D))


def kernel(x, edge_index, Wl0, Wr0, b0, Wl1, Wr1, b1, Wl2, Wr2, b2):
    pad = EPAD - E
    src = jnp.concatenate(
        [edge_index[0].astype(jnp.int32), jnp.zeros((pad,), jnp.int32)]
    ).reshape(NW, NCHUNK, K)
    dst = jnp.concatenate(
        [edge_index[1].astype(jnp.int32), jnp.full((pad,), N, jnp.int32)]
    ).reshape(NW, NCHUNK, K)
    sd = jnp.stack([src, dst], axis=2)
    z = jnp.zeros((ZR, D), jnp.float32)
    ones = jnp.ones((K, D), jnp.float32)

    cnts = _sc_cnt(dst, z, ones)
    parts = _sc_agg(x, sd, z)
    h = _tc_layer(parts, cnts, x, Wl0, Wr0, b0, act=True)
    parts = _sc_agg(h, sd, z)
    h = _tc_layer(parts, cnts, h, Wl1, Wr1, b1, act=True)
    parts = _sc_agg(h, sd, z)
    return _tc_layer(parts, cnts, h, Wl2, Wr2, b2, act=False)


# speedup vs baseline: 1.0174x; 1.0174x over previous
import functools

import jax
import jax.numpy as jnp
from jax import lax
from jax.experimental import pallas as pl
from jax.experimental.pallas import tpu as pltpu
from jax.experimental.pallas import tpu_sc as plsc

N = 10000
E = 320000
D = 128
NC = 2
NS = 16
NW = NC * NS
K = 64
EPAD = 327680
EPT = EPAD // NW
NCHUNK = EPT // K
NHALF = 2
HCHUNK = NCHUNK // NHALF
HPAIR = HCHUNK // 2
NA = 10240
ZT = NA // NS
ZB = 64
NZT = 10
ZR = N // NZT

_MESH = plsc.VectorSubcoreMesh(core_axis_name="c", subcore_axis_name="s",
                               num_cores=NC, num_subcores=NS)


def _sc_cnt_body(dst_hbm, z_hbm, ones_hbm,
                 out_cnt, cnt_acc, dst_v, ones_v):
    c = lax.axis_index("c")
    s = lax.axis_index("s")
    wid = s * NC + c
    stripe = pl.ds(s * ZR, ZR)

    pltpu.sync_copy(z_hbm, ones_v)

    def zbody(i, carry):
        pltpu.sync_copy(ones_v.at[pl.ds(0, ZB)],
                        cnt_acc.at[pl.ds(s * ZT + i * ZB, ZB)])
        return carry

    lax.fori_loop(0, ZT // ZB, zbody, 0)
    pltpu.sync_copy(ones_hbm, ones_v)
    pltpu.sync_copy(dst_hbm.at[wid], dst_v)
    plsc.subcore_barrier()

    def body(j, carry):
        pltpu.sync_copy(ones_v, cnt_acc.at[dst_v.at[j]], add=True)
        return carry

    lax.fori_loop(0, NCHUNK, body, 0)
    plsc.subcore_barrier()

    @pl.when(s < NZT)
    def _drain():
        pltpu.sync_copy(cnt_acc.at[stripe], out_cnt.at[c, stripe])


def _sc_body(h_hbm, src_hbm, dst_hbm, z_hbm,
             out_rows, acc, src_v, dst_v, rows0, rows1, sem0, sem1):
    c = lax.axis_index("c")
    s = lax.axis_index("s")
    wid = s * NC + c
    stripe = pl.ds(s * ZR, ZR)

    pltpu.sync_copy(z_hbm, rows0)

    def zbody(i, carry):
        pltpu.sync_copy(rows0.at[pl.ds(0, ZB)],
                        acc.at[pl.ds(s * ZT + i * ZB, ZB)])
        return carry

    lax.fori_loop(0, ZT // ZB, zbody, 0)
    plsc.subcore_barrier()

    for half in range(NHALF):
        pltpu.sync_copy(src_hbm.at[wid, half], src_v)
        pltpu.sync_copy(dst_hbm.at[wid, half], dst_v)
        pltpu.async_copy(h_hbm.at[src_v.at[0]], rows0, sem0)

        def pair(p, carry):
            j0 = 2 * p
            pltpu.async_copy(h_hbm.at[src_v.at[j0 + 1]], rows1, sem1)
            pltpu.make_async_copy(h_hbm.at[src_v.at[j0]], rows0, sem0).wait()
            pltpu.sync_copy(rows0, acc.at[dst_v.at[j0]], add=True)
            pltpu.async_copy(h_hbm.at[src_v.at[j0 + 2]], rows0, sem0)
            pltpu.make_async_copy(h_hbm.at[src_v.at[j0 + 1]], rows1,
                                  sem1).wait()
            pltpu.sync_copy(rows1, acc.at[dst_v.at[j0 + 1]], add=True)
            return carry

        lax.fori_loop(0, HPAIR - 1, pair, 0)
        jlast = HCHUNK - 2
        pltpu.async_copy(h_hbm.at[src_v.at[jlast + 1]], rows1, sem1)
        pltpu.make_async_copy(h_hbm.at[src_v.at[jlast]], rows0, sem0).wait()
        pltpu.sync_copy(rows0, acc.at[dst_v.at[jlast]], add=True)
        pltpu.make_async_copy(h_hbm.at[src_v.at[jlast + 1]], rows1,
                              sem1).wait()
        pltpu.sync_copy(rows1, acc.at[dst_v.at[jlast + 1]], add=True)
    plsc.subcore_barrier()

    @pl.when(s < NZT)
    def _drain():
        pltpu.sync_copy(acc.at[stripe], out_rows.at[c, stripe])


_sc_cnt = pl.kernel(
    _sc_cnt_body,
    out_type=jax.ShapeDtypeStruct((NC, N, D), jnp.float32),
    mesh=_MESH,
    scratch_types=[
        pltpu.VMEM_SHARED((NA, D), jnp.float32),
        pltpu.VMEM((NCHUNK, K), jnp.int32),
        pltpu.VMEM((K, D), jnp.float32),
    ],
)

_sc_agg = pl.kernel(
    _sc_body,
    out_type=jax.ShapeDtypeStruct((NC, N, D), jnp.float32),
    mesh=_MESH,
    scratch_types=[
        pltpu.VMEM_SHARED((NA, D), jnp.float32),
        pltpu.VMEM((HCHUNK, K), jnp.int32),
        pltpu.VMEM((HCHUNK, K), jnp.int32),
        pltpu.VMEM((K, D), jnp.float32),
        pltpu.VMEM((K, D), jnp.float32),
        pltpu.SemaphoreType.DMA,
        pltpu.SemaphoreType.DMA,
    ],
)


BN = 1000


def _tc_body(p0, p1, c0, c1, h, wl, wr, b, o, *, act):
    cnt = c0[:, 0:1] + c1[:, 0:1]
    inv = 1.0 / jnp.maximum(cnt, 1.0)
    agg = (p0[:, :] + p1[:, :]) * inv
    y = (jnp.dot(agg, wl[:, :], preferred_element_type=jnp.float32)
         + jnp.dot(h[:, :], wr[:, :], preferred_element_type=jnp.float32)
         + b[:, :])
    o[:, :] = jnp.tanh(y) if act else y


def _tc_layer(parts, cnts, h, Wl, Wr, b, act):
    return pl.pallas_call(
        functools.partial(_tc_body, act=act),
        grid=(N // BN,),
        in_specs=[
            pl.BlockSpec((BN, D), lambda i: (i, 0)),
            pl.BlockSpec((BN, D), lambda i: (i, 0)),
            pl.BlockSpec((BN, D), lambda i: (i, 0)),
            pl.BlockSpec((BN, D), lambda i: (i, 0)),
            pl.BlockSpec((BN, D), lambda i: (i, 0)),
            pl.BlockSpec((D, D), lambda i: (0, 0)),
            pl.BlockSpec((D, D), lambda i: (0, 0)),
            pl.BlockSpec((1, D), lambda i: (0, 0)),
        ],
        out_specs=pl.BlockSpec((BN, D), lambda i: (i, 0)),
        out_shape=jax.ShapeDtypeStruct((N, D), jnp.float32),
    )(parts[0], parts[1], cnts[0], cnts[1], h, Wl, Wr, b.reshape(1, D))


def kernel(x, edge_index, Wl0, Wr0, b0, Wl1, Wr1, b1, Wl2, Wr2, b2):
    pad = EPAD - E
    srcf = jnp.concatenate(
        [edge_index[0].astype(jnp.int32), jnp.zeros((pad,), jnp.int32)])
    dstf = jnp.concatenate(
        [edge_index[1].astype(jnp.int32), jnp.full((pad,), N, jnp.int32)])
    src = srcf.reshape(NW, NHALF, HCHUNK, K)
    dst = dstf.reshape(NW, NHALF, HCHUNK, K)
    dstc = dstf.reshape(NW, NCHUNK, K)

    z = jnp.zeros((K, D), jnp.float32)
    ones = jnp.ones((K, D), jnp.float32)

    cnts = _sc_cnt(dstc, z, ones)
    parts = _sc_agg(x, src, dst, z)
    h = _tc_layer(parts, cnts, x, Wl0, Wr0, b0, act=True)
    parts = _sc_agg(h, src, dst, z)
    h = _tc_layer(parts, cnts, h, Wl1, Wr1, b1, act=True)
    parts = _sc_agg(h, src, dst, z)
    return _tc_layer(parts, cnts, h, Wl2, Wr2, b2, act=False)


# speedup vs baseline: 2.7840x; 2.7364x over previous
import functools

import jax
import jax.numpy as jnp
from jax import lax
from jax.experimental import pallas as pl
from jax.experimental.pallas import tpu as pltpu
from jax.experimental.pallas import tpu_sc as plsc

N = 10000
E = 320000
D = 128
NC = 2
NS = 16
NW = NC * NS
K = 64
EPAD = 327680
EPT = EPAD // NW
NCHUNK = EPT // K
NHALF = 2
HCHUNK = NCHUNK // NHALF
HPAIR = HCHUNK // 2
NA = 10240
ZT = NA // NS
ZB = 64
NZT = 10
ZR = N // NZT

_MESH = plsc.VectorSubcoreMesh(core_axis_name="c", subcore_axis_name="s",
                               num_cores=NC, num_subcores=NS)


def _sc_cnt_body(dst_hbm, z_hbm, ones_hbm,
                 out_cnt, cnt_acc, dst_v, ones_v):
    c = lax.axis_index("c")
    s = lax.axis_index("s")
    wid = s * NC + c
    stripe = pl.ds(s * ZR, ZR)

    pltpu.sync_copy(z_hbm, ones_v)

    def zbody(i, carry):
        pltpu.sync_copy(ones_v.at[pl.ds(0, ZB)],
                        cnt_acc.at[pl.ds(s * ZT + i * ZB, ZB)])
        return carry

    lax.fori_loop(0, ZT // ZB, zbody, 0)
    pltpu.sync_copy(ones_hbm, ones_v)
    pltpu.sync_copy(dst_hbm.at[wid], dst_v)
    plsc.subcore_barrier()

    def body(j, carry):
        pltpu.sync_copy(ones_v, cnt_acc.at[dst_v.at[j]], add=True)
        return carry

    lax.fori_loop(0, NCHUNK, body, 0)
    plsc.subcore_barrier()

    @pl.when(s < NZT)
    def _drain():
        pltpu.sync_copy(cnt_acc.at[stripe], out_cnt.at[c, stripe])


def _sc_body(h_hbm, src_hbm, dst_hbm, z_hbm,
             out_rows, acc, src_v, dst_v, rows0, rows1, sem0, sem1):
    c = lax.axis_index("c")
    s = lax.axis_index("s")
    wid = s * NC + c
    stripe = pl.ds(s * ZR, ZR)

    pltpu.sync_copy(z_hbm, rows0)

    def zbody(i, carry):
        pltpu.sync_copy(rows0.at[pl.ds(0, ZB)],
                        acc.at[pl.ds(s * ZT + i * ZB, ZB)])
        return carry

    lax.fori_loop(0, ZT // ZB, zbody, 0)
    plsc.subcore_barrier()

    for half in range(NHALF):
        pltpu.sync_copy(src_hbm.at[wid, half], src_v)
        pltpu.sync_copy(dst_hbm.at[wid, half], dst_v)
        pltpu.async_copy(h_hbm.at[src_v.at[0]], rows0, sem0)

        def pair(p, carry):
            j0 = 2 * p
            pltpu.async_copy(h_hbm.at[src_v.at[j0 + 1]], rows1, sem1)
            pltpu.make_async_copy(h_hbm.at[src_v.at[j0]], rows0, sem0).wait()
            pltpu.sync_copy(rows0, acc.at[dst_v.at[j0]], add=True)
            pltpu.async_copy(h_hbm.at[src_v.at[j0 + 2]], rows0, sem0)
            pltpu.make_async_copy(h_hbm.at[src_v.at[j0 + 1]], rows1,
                                  sem1).wait()
            pltpu.sync_copy(rows1, acc.at[dst_v.at[j0 + 1]], add=True)
            return carry

        lax.fori_loop(0, HPAIR - 1, pair, 0)
        jlast = HCHUNK - 2
        pltpu.async_copy(h_hbm.at[src_v.at[jlast + 1]], rows1, sem1)
        pltpu.make_async_copy(h_hbm.at[src_v.at[jlast]], rows0, sem0).wait()
        pltpu.sync_copy(rows0, acc.at[dst_v.at[jlast]], add=True)
        pltpu.make_async_copy(h_hbm.at[src_v.at[jlast + 1]], rows1,
                              sem1).wait()
        pltpu.sync_copy(rows1, acc.at[dst_v.at[jlast + 1]], add=True)
    plsc.subcore_barrier()

    @pl.when(s < NZT)
    def _drain():
        pltpu.sync_copy(acc.at[stripe], out_rows.at[c, stripe])


_sc_cnt = pl.kernel(
    _sc_cnt_body,
    out_type=jax.ShapeDtypeStruct((NC, N, D), jnp.float32),
    mesh=_MESH,
    scratch_types=[
        pltpu.VMEM_SHARED((NA, D), jnp.float32),
        pltpu.VMEM((NCHUNK, K), jnp.int32),
        pltpu.VMEM((K, D), jnp.float32),
    ],
)

_sc_agg = pl.kernel(
    _sc_body,
    out_type=jax.ShapeDtypeStruct((NC, N, D), jnp.float32),
    mesh=_MESH,
    scratch_types=[
        pltpu.VMEM_SHARED((NA, D), jnp.float32),
        pltpu.VMEM((HCHUNK, K), jnp.int32),
        pltpu.VMEM((HCHUNK, K), jnp.int32),
        pltpu.VMEM((K, D), jnp.float32),
        pltpu.VMEM((K, D), jnp.float32),
        pltpu.SemaphoreType.DMA,
        pltpu.SemaphoreType.DMA,
    ],
)


BN = 1000


def _tc_body(p0, p1, c0, c1, h, wl, wr, b, o, *, act):
    cnt = c0[:, 0:1] + c1[:, 0:1]
    inv = 1.0 / jnp.maximum(cnt, 1.0)
    agg = (p0[:, :] + p1[:, :]) * inv
    y = (jnp.dot(agg, wl[:, :], preferred_element_type=jnp.float32)
         + jnp.dot(h[:, :], wr[:, :], preferred_element_type=jnp.float32)
         + b[:, :])
    o[:, :] = jnp.tanh(y) if act else y


def _tc_layer(parts, cnts, h, Wl, Wr, b, act):
    return pl.pallas_call(
        functools.partial(_tc_body, act=act),
        grid=(N // BN,),
        in_specs=[
            pl.BlockSpec((BN, D), lambda i: (i, 0)),
            pl.BlockSpec((BN, D), lambda i: (i, 0)),
            pl.BlockSpec((BN, D), lambda i: (i, 0)),
            pl.BlockSpec((BN, D), lambda i: (i, 0)),
            pl.BlockSpec((BN, D), lambda i: (i, 0)),
            pl.BlockSpec((D, D), lambda i: (0, 0)),
            pl.BlockSpec((D, D), lambda i: (0, 0)),
            pl.BlockSpec((1, D), lambda i: (0, 0)),
        ],
        out_specs=pl.BlockSpec((BN, D), lambda i: (i, 0)),
        out_shape=jax.ShapeDtypeStruct((N, D), jnp.float32),
    )(parts[0], parts[1], cnts[0], cnts[1], h, Wl, Wr, b.reshape(1, D))


def kernel(x, edge_index, Wl0, Wr0, b0, Wl1, Wr1, b1, Wl2, Wr2, b2):
    pad = EPAD - E
    padi = jnp.arange(pad, dtype=jnp.int32)
    srcf = jnp.concatenate(
        [edge_index[0].astype(jnp.int32), padi % N])
    dstf = jnp.concatenate(
        [edge_index[1].astype(jnp.int32), N + padi % (NA - N)])
    src = srcf.reshape(NW, NHALF, HCHUNK, K)
    dst = dstf.reshape(NW, NHALF, HCHUNK, K)
    dstc = dstf.reshape(NW, NCHUNK, K)

    z = jnp.zeros((K, D), jnp.float32)
    ones = jnp.ones((K, D), jnp.float32)

    cnts = _sc_cnt(dstc, z, ones)
    parts = _sc_agg(x, src, dst, z)
    h = _tc_layer(parts, cnts, x, Wl0, Wr0, b0, act=True)
    parts = _sc_agg(h, src, dst, z)
    h = _tc_layer(parts, cnts, h, Wl1, Wr1, b1, act=True)
    parts = _sc_agg(h, src, dst, z)
    return _tc_layer(parts, cnts, h, Wl2, Wr2, b2, act=False)


# speedup vs baseline: 3.0200x; 1.0848x over previous
import functools

import jax
import jax.numpy as jnp
from jax import lax
from jax.experimental import pallas as pl
from jax.experimental.pallas import tpu as pltpu
from jax.experimental.pallas import tpu_sc as plsc

N = 10000
E = 320000
D = 128
NC = 2
NS = 16
NW = NC * NS
K = 128
EPAD = 327680
EPT = EPAD // NW
NCHUNK = EPT // K
NQ = 4
QCHUNK = NCHUNK // NQ
QPAIR = QCHUNK // 2
NA = 10240
ZT = NA // NS
NZB = ZT // K
NDT = 10

_MESH = plsc.VectorSubcoreMesh(core_axis_name="c", subcore_axis_name="s",
                               num_cores=NC, num_subcores=NS)


def _zero_acc(z_hbm, zidx_hbm, acc, zbuf, zv, s):
    pltpu.sync_copy(z_hbm, zbuf)
    pltpu.sync_copy(zidx_hbm.at[s], zv)

    def zbody(i, carry):
        pltpu.sync_copy(zbuf, acc.at[zv.at[i]])
        return carry

    lax.fori_loop(0, NZB, zbody, 0)


def _sc_body(h_hbm, src_hbm, dst_hbm, z_hbm, zidx_hbm,
             out_rows, acc, src_v, dst_v, zv, rows0, rows1, sem0, sem1):
    c = lax.axis_index("c")
    s = lax.axis_index("s")
    wid = s * NC + c
    _zero_acc(z_hbm, zidx_hbm, acc, rows0, zv, s)
    plsc.subcore_barrier()

    for q in range(NQ):
        pltpu.sync_copy(src_hbm.at[wid, q], src_v)
        pltpu.sync_copy(dst_hbm.at[wid, q], dst_v)
        pltpu.async_copy(h_hbm.at[src_v.at[0]], rows0, sem0)

        def pair(p, carry):
            j0 = 2 * p
            pltpu.async_copy(h_hbm.at[src_v.at[j0 + 1]], rows1, sem1)
            pltpu.make_async_copy(h_hbm.at[src_v.at[j0]], rows0, sem0).wait()
            pltpu.sync_copy(rows0, acc.at[dst_v.at[j0]], add=True)
            pltpu.async_copy(h_hbm.at[src_v.at[j0 + 2]], rows0, sem0)
            pltpu.make_async_copy(h_hbm.at[src_v.at[j0 + 1]], rows1,
                                  sem1).wait()
            pltpu.sync_copy(rows1, acc.at[dst_v.at[j0 + 1]], add=True)
            return carry

        lax.fori_loop(0, QPAIR - 1, pair, 0)
        jlast = QCHUNK - 2
        pltpu.async_copy(h_hbm.at[src_v.at[jlast + 1]], rows1, sem1)
        pltpu.make_async_copy(h_hbm.at[src_v.at[jlast]], rows0, sem0).wait()
        pltpu.sync_copy(rows0, acc.at[dst_v.at[jlast]], add=True)
        pltpu.make_async_copy(h_hbm.at[src_v.at[jlast + 1]], rows1,
                              sem1).wait()
        pltpu.sync_copy(rows1, acc.at[dst_v.at[jlast + 1]], add=True)
    plsc.subcore_barrier()

    @pl.when(s < NDT)
    def _drain():
        stripe = pl.ds(s * (N // NDT), N // NDT)
        pltpu.sync_copy(acc.at[stripe], out_rows.at[c, stripe])


_sc_agg = pl.kernel(
    _sc_body,
    out_type=jax.ShapeDtypeStruct((NC, N, D), jnp.float32),
    mesh=_MESH,
    scratch_types=[
        pltpu.VMEM_SHARED((NA, D), jnp.float32),
        pltpu.VMEM((QCHUNK, K), jnp.int32),
        pltpu.VMEM((QCHUNK, K), jnp.int32),
        pltpu.VMEM((NZB, K), jnp.int32),
        pltpu.VMEM((K, D), jnp.float32),
        pltpu.VMEM((K, D), jnp.float32),
        pltpu.SemaphoreType.DMA,
        pltpu.SemaphoreType.DMA,
    ],
)


def _sc_cnt_body(dst_hbm, z_hbm, zidx_hbm, ones_hbm,
                 out_cnt, cnt_acc, dst_v, zv, ones_v):
    c = lax.axis_index("c")
    s = lax.axis_index("s")
    wid = s * NC + c
    _zero_acc(z_hbm, zidx_hbm, cnt_acc, ones_v, zv, s)
    pltpu.sync_copy(ones_hbm, ones_v)
    pltpu.sync_copy(dst_hbm.at[wid], dst_v)
    plsc.subcore_barrier()

    def body(j, carry):
        pltpu.sync_copy(ones_v, cnt_acc.at[dst_v.at[j]], add=True)
        return carry

    lax.fori_loop(0, NCHUNK, body, 0)
    plsc.subcore_barrier()

    @pl.when(s < NDT)
    def _drain():
        stripe = pl.ds(s * (N // NDT), N // NDT)
        pltpu.sync_copy(cnt_acc.at[stripe], out_cnt.at[c, stripe])


_sc_cnt = pl.kernel(
    _sc_cnt_body,
    out_type=jax.ShapeDtypeStruct((NC, N, D), jnp.float32),
    mesh=_MESH,
    scratch_types=[
        pltpu.VMEM_SHARED((NA, D), jnp.float32),
        pltpu.VMEM((NCHUNK, K), jnp.int32),
        pltpu.VMEM((NZB, K), jnp.int32),
        pltpu.VMEM((K, D), jnp.float32),
    ],
)


BN = 1000


def _tc_body(p0, p1, c0, c1, h, wl, wr, b, o, *, act):
    cnt = c0[:, 0:1] + c1[:, 0:1]
    inv = 1.0 / jnp.maximum(cnt, 1.0)
    agg = (p0[:, :] + p1[:, :]) * inv
    y = (jnp.dot(agg, wl[:, :], preferred_element_type=jnp.float32)
         + jnp.dot(h[:, :], wr[:, :], preferred_element_type=jnp.float32)
         + b[:, :])
    o[:, :] = jnp.tanh(y) if act else y


def _tc_layer(parts, cnts, h, Wl, Wr, b, act):
    return pl.pallas_call(
        functools.partial(_tc_body, act=act),
        grid=(N // BN,),
        in_specs=[
            pl.BlockSpec((BN, D), lambda i: (i, 0)),
            pl.BlockSpec((BN, D), lambda i: (i, 0)),
            pl.BlockSpec((BN, D), lambda i: (i, 0)),
            pl.BlockSpec((BN, D), lambda i: (i, 0)),
            pl.BlockSpec((BN, D), lambda i: (i, 0)),
            pl.BlockSpec((D, D), lambda i: (0, 0)),
            pl.BlockSpec((D, D), lambda i: (0, 0)),
            pl.BlockSpec((1, D), lambda i: (0, 0)),
        ],
        out_specs=pl.BlockSpec((BN, D), lambda i: (i, 0)),
        out_shape=jax.ShapeDtypeStruct((N, D), jnp.float32),
    )(parts[0], parts[1], cnts[0], cnts[1], h, Wl, Wr, b.reshape(1, D))


def kernel(x, edge_index, Wl0, Wr0, b0, Wl1, Wr1, b1, Wl2, Wr2, b2):
    pad = EPAD - E
    padi = jnp.arange(pad, dtype=jnp.int32)
    srcf = jnp.concatenate([edge_index[0].astype(jnp.int32), padi % N])
    dstf = jnp.concatenate([edge_index[1].astype(jnp.int32),
                            N + padi % (NA - N)])
    src = srcf.reshape(NW, NQ, QCHUNK, K)
    dst = dstf.reshape(NW, NQ, QCHUNK, K)
    dstc = dstf.reshape(NW, NCHUNK, K)
    z = jnp.zeros((K, D), jnp.float32)
    ones = jnp.ones((K, D), jnp.float32)
    zidx = jnp.arange(NA, dtype=jnp.int32).reshape(NS, NZB, K)

    cnts = _sc_cnt(dstc, z, zidx, ones)
    parts = _sc_agg(x, src, dst, z, zidx)
    h = _tc_layer(parts, cnts, x, Wl0, Wr0, b0, act=True)
    parts = _sc_agg(h, src, dst, z, zidx)
    h = _tc_layer(parts, cnts, h, Wl1, Wr1, b1, act=True)
    parts = _sc_agg(h, src, dst, z, zidx)
    return _tc_layer(parts, cnts, h, Wl2, Wr2, b2, act=False)


# speedup vs baseline: 3.3720x; 1.1166x over previous
import functools

import jax
import jax.numpy as jnp
from jax import lax
from jax.experimental import pallas as pl
from jax.experimental.pallas import tpu as pltpu
from jax.experimental.pallas import tpu_sc as plsc

N = 10000
E = 320000
D = 128
NC = 2
NS = 16
NW = NC * NS
K = 128
EPAD = 327680
EPT = EPAD // NW
NCHUNK = EPT // K
NQ = 4
QCHUNK = NCHUNK // NQ
QPAIR = QCHUNK // 2
NA = 10240
ZT = NA // NS
NZB = ZT // K
NDT = 10

_MESH = plsc.VectorSubcoreMesh(core_axis_name="c", subcore_axis_name="s",
                               num_cores=NC, num_subcores=NS)


def _zero_acc(z_hbm, zidx_hbm, acc, zbuf, zv, s):
    pltpu.sync_copy(z_hbm, zbuf)
    pltpu.sync_copy(zidx_hbm.at[s], zv)

    def zbody(i, carry):
        pltpu.sync_copy(zbuf, acc.at[zv.at[i]])
        return carry

    lax.fori_loop(0, NZB, zbody, 0)


def _sc_body(h_hbm, src_hbm, dst_hbm, z_hbm, zidx_hbm,
             out_rows, acc, src_v, dst_v, zv, rows0, rows1, sem0, sem1):
    c = lax.axis_index("c")
    s = lax.axis_index("s")
    wid = s * NC + c
    _zero_acc(z_hbm, zidx_hbm, acc, rows0, zv, s)
    plsc.subcore_barrier()

    for q in range(NQ):
        pltpu.sync_copy(src_hbm.at[wid, q], src_v)
        pltpu.sync_copy(dst_hbm.at[wid, q], dst_v)
        pltpu.async_copy(h_hbm.at[src_v.at[0]], rows0, sem0)

        def pair(p, carry):
            j0 = 2 * p
            pltpu.async_copy(h_hbm.at[src_v.at[j0 + 1]], rows1, sem1)
            pltpu.make_async_copy(h_hbm.at[src_v.at[j0]], rows0, sem0).wait()
            pltpu.sync_copy(rows0, acc.at[dst_v.at[j0]], add=True)
            pltpu.async_copy(h_hbm.at[src_v.at[j0 + 2]], rows0, sem0)
            pltpu.make_async_copy(h_hbm.at[src_v.at[j0 + 1]], rows1,
                                  sem1).wait()
            pltpu.sync_copy(rows1, acc.at[dst_v.at[j0 + 1]], add=True)
            return carry

        lax.fori_loop(0, QPAIR - 1, pair, 0)
        jlast = QCHUNK - 2
        pltpu.async_copy(h_hbm.at[src_v.at[jlast + 1]], rows1, sem1)
        pltpu.make_async_copy(h_hbm.at[src_v.at[jlast]], rows0, sem0).wait()
        pltpu.sync_copy(rows0, acc.at[dst_v.at[jlast]], add=True)
        pltpu.make_async_copy(h_hbm.at[src_v.at[jlast + 1]], rows1,
                              sem1).wait()
        pltpu.sync_copy(rows1, acc.at[dst_v.at[jlast + 1]], add=True)
    plsc.subcore_barrier()

    @pl.when(s < NDT)
    def _drain():
        stripe = pl.ds(s * (N // NDT), N // NDT)
        pltpu.sync_copy(acc.at[stripe], out_rows.at[c, stripe])


_sc_agg = pl.kernel(
    _sc_body,
    out_type=jax.ShapeDtypeStruct((NC, N, D), jnp.float32),
    mesh=_MESH,
    scratch_types=[
        pltpu.VMEM_SHARED((NA, D), jnp.float32),
        pltpu.VMEM((QCHUNK, K), jnp.int32),
        pltpu.VMEM((QCHUNK, K), jnp.int32),
        pltpu.VMEM((NZB, K), jnp.int32),
        pltpu.VMEM((K, D), jnp.float32),
        pltpu.VMEM((K, D), jnp.float32),
        pltpu.SemaphoreType.DMA,
        pltpu.SemaphoreType.DMA,
    ],
)


NV = EPT // 16
HC = NA // 128


def _sc_cnt_body(dst_hbm, hidx_hbm, out_cnt, hist_sp, dst_v, hist, hidx_v):
    c = lax.axis_index("c")
    s = lax.axis_index("s")
    wid = s * NC + c
    zrow = jnp.zeros((16,), jnp.float32)

    def zb(i, carry):
        hist[pl.ds(i * 16, 16)] = zrow
        return carry

    lax.fori_loop(0, NV, zb, 0)
    pltpu.sync_copy(hist.at[pl.ds(0, NA // NS)],
                    hist_sp.at[pl.ds(s * (NA // NS), NA // NS)])
    pltpu.sync_copy(dst_hbm.at[wid], dst_v)
    pltpu.sync_copy(hidx_hbm, hidx_v)
    ones = jnp.ones((16,), jnp.float32)
    plsc.subcore_barrier()

    def hb(e, carry):
        v = dst_v[pl.ds(e * 16, 16)]
        plsc.addupdate_scatter(hist, [v], ones)
        return carry

    lax.fori_loop(0, NV, hb, 0)

    def rb(i, carry):
        pltpu.sync_copy(hist.at[pl.ds(i * 128, 128)],
                        hist_sp.at[hidx_v.at[i]], add=True)
        return carry

    lax.fori_loop(0, HC, rb, 0)
    plsc.subcore_barrier()

    @pl.when(s < 8)
    def _drain():
        stripe = pl.ds(s * (NA // 8), NA // 8)
        pltpu.sync_copy(hist_sp.at[stripe], out_cnt.at[c, stripe])


_sc_cnt = pl.kernel(
    _sc_cnt_body,
    out_type=jax.ShapeDtypeStruct((NC, NA), jnp.float32),
    mesh=_MESH,
    compiler_params=pltpu.CompilerParams(needs_layout_passes=False),
    scratch_types=[
        pltpu.VMEM_SHARED((NA,), jnp.float32),
        pltpu.VMEM((EPT,), jnp.int32),
        pltpu.VMEM((NA,), jnp.float32),
        pltpu.VMEM((HC, 128), jnp.int32),
    ],
)


BN = 1000


def _tc_body(p0, p1, c0, c1, h, wl, wr, b, o, *, act):
    cnt = c0[:, :] + c1[:, :]
    inv = 1.0 / jnp.maximum(cnt, 1.0)
    agg = (p0[:, :] + p1[:, :]) * inv
    y = (jnp.dot(agg, wl[:, :], preferred_element_type=jnp.float32)
         + jnp.dot(h[:, :], wr[:, :], preferred_element_type=jnp.float32)
         + b[:, :])
    o[:, :] = jnp.tanh(y) if act else y


def _tc_layer(parts, cnts, h, Wl, Wr, b, act):
    return pl.pallas_call(
        functools.partial(_tc_body, act=act),
        grid=(N // BN,),
        in_specs=[
            pl.BlockSpec((BN, D), lambda i: (i, 0)),
            pl.BlockSpec((BN, D), lambda i: (i, 0)),
            pl.BlockSpec((BN, 1), lambda i: (i, 0)),
            pl.BlockSpec((BN, 1), lambda i: (i, 0)),
            pl.BlockSpec((BN, D), lambda i: (i, 0)),
            pl.BlockSpec((D, D), lambda i: (0, 0)),
            pl.BlockSpec((D, D), lambda i: (0, 0)),
            pl.BlockSpec((1, D), lambda i: (0, 0)),
        ],
        out_specs=pl.BlockSpec((BN, D), lambda i: (i, 0)),
        out_shape=jax.ShapeDtypeStruct((N, D), jnp.float32),
    )(parts[0], parts[1], cnts[0], cnts[1], h, Wl, Wr, b.reshape(1, D))


def kernel(x, edge_index, Wl0, Wr0, b0, Wl1, Wr1, b1, Wl2, Wr2, b2):
    pad = EPAD - E
    padi = jnp.arange(pad, dtype=jnp.int32)
    srcf = jnp.concatenate([edge_index[0].astype(jnp.int32), padi % N])
    dstf = jnp.concatenate([edge_index[1].astype(jnp.int32),
                            N + padi % (NA - N)])
    src = srcf.reshape(NW, NQ, QCHUNK, K)
    dst = dstf.reshape(NW, NQ, QCHUNK, K)
    dstc = dstf.reshape(NW, EPT)
    z = jnp.zeros((K, D), jnp.float32)
    zidx = jnp.arange(NA, dtype=jnp.int32).reshape(NS, NZB, K)
    hidx = jnp.arange(NA, dtype=jnp.int32).reshape(HC, 128)

    cnts = _sc_cnt(dstc, hidx)
    cnts = cnts[:, :N].reshape(NC, N, 1)
    parts = _sc_agg(x, src, dst, z, zidx)
    h = _tc_layer(parts, cnts, x, Wl0, Wr0, b0, act=True)
    parts = _sc_agg(h, src, dst, z, zidx)
    h = _tc_layer(parts, cnts, h, Wl1, Wr1, b1, act=True)
    parts = _sc_agg(h, src, dst, z, zidx)
    return _tc_layer(parts, cnts, h, Wl2, Wr2, b2, act=False)


# speedup vs baseline: 3.3870x; 1.0044x over previous
import functools

import jax
import jax.numpy as jnp
from jax import lax
from jax.experimental import pallas as pl
from jax.experimental.pallas import tpu as pltpu
from jax.experimental.pallas import tpu_sc as plsc

N = 10000
E = 320000
D = 128
NC = 2
NS = 16
NW = NC * NS
K = 128
EPAD = 327680
EPT = EPAD // NW
NCHUNK = EPT // K
NQ = 4
QCHUNK = NCHUNK // NQ
QPAIR = QCHUNK // 2
NA = 10240
ZT = NA // NS
NZB = ZT // K
NDT = 10

_MESH = plsc.VectorSubcoreMesh(core_axis_name="c", subcore_axis_name="s",
                               num_cores=NC, num_subcores=NS)


def _zero_acc(z_hbm, zidx_hbm, acc, zbuf, zv, s):
    pltpu.sync_copy(z_hbm, zbuf)
    pltpu.sync_copy(zidx_hbm.at[s], zv)

    def zbody(i, carry):
        pltpu.sync_copy(zbuf, acc.at[zv.at[i]])
        return carry

    lax.fori_loop(0, NZB, zbody, 0)


def _sc_body(h_hbm, src_hbm, dst_hbm, z_hbm, zidx_hbm,
             out_rows, acc, src_v, dst_v, zv, rows0, rows1, sem0, sem1):
    c = lax.axis_index("c")
    s = lax.axis_index("s")
    wid = s * NC + c
    _zero_acc(z_hbm, zidx_hbm, acc, rows0, zv, s)
    plsc.subcore_barrier()

    bufs = (rows0, rows1)
    sems = (sem0, sem1)

    def quarter(q, carry):
        pltpu.sync_copy(src_hbm.at[wid, q], src_v)
        pltpu.sync_copy(dst_hbm.at[wid, q], dst_v)
        pltpu.async_copy(h_hbm.at[src_v.at[0]], rows0, sem0)
        for j in range(QCHUNK):
            if j + 1 < QCHUNK:
                pltpu.async_copy(h_hbm.at[src_v.at[j + 1]],
                                 bufs[(j + 1) % 2], sems[(j + 1) % 2])
            pltpu.make_async_copy(h_hbm.at[src_v.at[j]], bufs[j % 2],
                                  sems[j % 2]).wait()
            pltpu.sync_copy(bufs[j % 2], acc.at[dst_v.at[j]], add=True)
        return carry

    lax.fori_loop(0, NQ, quarter, 0)
    plsc.subcore_barrier()

    @pl.when(s < NDT)
    def _drain():
        stripe = pl.ds(s * (N // NDT), N // NDT)
        pltpu.sync_copy(acc.at[stripe], out_rows.at[c, stripe])


_sc_agg = pl.kernel(
    _sc_body,
    out_type=jax.ShapeDtypeStruct((NC, N, D), jnp.float32),
    mesh=_MESH,
    scratch_types=[
        pltpu.VMEM_SHARED((NA, D), jnp.float32),
        pltpu.VMEM((QCHUNK, K), jnp.int32),
        pltpu.VMEM((QCHUNK, K), jnp.int32),
        pltpu.VMEM((NZB, K), jnp.int32),
        pltpu.VMEM((K, D), jnp.float32),
        pltpu.VMEM((K, D), jnp.float32),
        pltpu.SemaphoreType.DMA,
        pltpu.SemaphoreType.DMA,
    ],
)


NV = EPT // 16
HC = NA // 128


def _sc_cnt_body(dst_hbm, hidx_hbm, out_cnt, hist_sp, dst_v, hist, hidx_v):
    c = lax.axis_index("c")
    s = lax.axis_index("s")
    wid = s * NC + c
    zrow = jnp.zeros((16,), jnp.float32)

    def zb(i, carry):
        hist[pl.ds(i * 16, 16)] = zrow
        return carry

    lax.fori_loop(0, NV, zb, 0)
    pltpu.sync_copy(hist.at[pl.ds(0, NA // NS)],
                    hist_sp.at[pl.ds(s * (NA // NS), NA // NS)])
    pltpu.sync_copy(dst_hbm.at[wid], dst_v)
    pltpu.sync_copy(hidx_hbm, hidx_v)
    ones = jnp.ones((16,), jnp.float32)
    plsc.subcore_barrier()

    def hb(e, carry):
        v = dst_v[pl.ds(e * 16, 16)]
        plsc.addupdate_scatter(hist, [v], ones)
        return carry

    lax.fori_loop(0, NV, hb, 0)

    def rb(i, carry):
        pltpu.sync_copy(hist.at[pl.ds(i * 128, 128)],
                        hist_sp.at[hidx_v.at[i]], add=True)
        return carry

    lax.fori_loop(0, HC, rb, 0)
    plsc.subcore_barrier()

    @pl.when(s < 8)
    def _drain():
        stripe = pl.ds(s * (NA // 8), NA // 8)
        pltpu.sync_copy(hist_sp.at[stripe], out_cnt.at[c, stripe])


_sc_cnt = pl.kernel(
    _sc_cnt_body,
    out_type=jax.ShapeDtypeStruct((NC, NA), jnp.float32),
    mesh=_MESH,
    compiler_params=pltpu.CompilerParams(needs_layout_passes=False),
    scratch_types=[
        pltpu.VMEM_SHARED((NA,), jnp.float32),
        pltpu.VMEM((EPT,), jnp.int32),
        pltpu.VMEM((NA,), jnp.float32),
        pltpu.VMEM((HC, 128), jnp.int32),
    ],
)


BN = 1000


def _tc_body(p0, p1, c0, c1, h, wl, wr, b, o, *, act):
    cnt = c0[:, :] + c1[:, :]
    inv = 1.0 / jnp.maximum(cnt, 1.0)
    agg = (p0[:, :] + p1[:, :]) * inv
    y = (jnp.dot(agg, wl[:, :], preferred_element_type=jnp.float32)
         + jnp.dot(h[:, :], wr[:, :], preferred_element_type=jnp.float32)
         + b[:, :])
    o[:, :] = jnp.tanh(y) if act else y


def _tc_layer(parts, cnts, h, Wl, Wr, b, act):
    return pl.pallas_call(
        functools.partial(_tc_body, act=act),
        grid=(N // BN,),
        in_specs=[
            pl.BlockSpec((BN, D), lambda i: (i, 0)),
            pl.BlockSpec((BN, D), lambda i: (i, 0)),
            pl.BlockSpec((BN, 1), lambda i: (i, 0)),
            pl.BlockSpec((BN, 1), lambda i: (i, 0)),
            pl.BlockSpec((BN, D), lambda i: (i, 0)),
            pl.BlockSpec((D, D), lambda i: (0, 0)),
            pl.BlockSpec((D, D), lambda i: (0, 0)),
            pl.BlockSpec((1, D), lambda i: (0, 0)),
        ],
        out_specs=pl.BlockSpec((BN, D), lambda i: (i, 0)),
        out_shape=jax.ShapeDtypeStruct((N, D), jnp.float32),
    )(parts[0], parts[1], cnts[0], cnts[1], h, Wl, Wr, b.reshape(1, D))


def kernel(x, edge_index, Wl0, Wr0, b0, Wl1, Wr1, b1, Wl2, Wr2, b2):
    pad = EPAD - E
    padi = jnp.arange(pad, dtype=jnp.int32)
    srcf = jnp.concatenate([edge_index[0].astype(jnp.int32), padi % N])
    dstf = jnp.concatenate([edge_index[1].astype(jnp.int32),
                            N + padi % (NA - N)])
    src = srcf.reshape(NW, NQ, QCHUNK, K)
    dst = dstf.reshape(NW, NQ, QCHUNK, K)
    dstc = dstf.reshape(NW, EPT)
    z = jnp.zeros((K, D), jnp.float32)
    zidx = jnp.arange(NA, dtype=jnp.int32).reshape(NS, NZB, K)
    hidx = jnp.arange(NA, dtype=jnp.int32).reshape(HC, 128)

    cnts = _sc_cnt(dstc, hidx)
    cnts = cnts[:, :N].reshape(NC, N, 1)
    parts = _sc_agg(x, src, dst, z, zidx)
    h = _tc_layer(parts, cnts, x, Wl0, Wr0, b0, act=True)
    parts = _sc_agg(h, src, dst, z, zidx)
    h = _tc_layer(parts, cnts, h, Wl1, Wr1, b1, act=True)
    parts = _sc_agg(h, src, dst, z, zidx)
    return _tc_layer(parts, cnts, h, Wl2, Wr2, b2, act=False)


# speedup vs baseline: 3.3885x; 1.0005x over previous
import functools

import jax
import jax.numpy as jnp
from jax import lax
from jax.experimental import pallas as pl
from jax.experimental.pallas import tpu as pltpu
from jax.experimental.pallas import tpu_sc as plsc

N = 10000
E = 320000
D = 128
NC = 2
NS = 16
NW = NC * NS
K = 128
EPAD = 327680
EPT = EPAD // NW
NCHUNK = EPT // K
NQ = 4
QCHUNK = NCHUNK // NQ
NA = 10240
ZT = NA // NS
NZB = ZT // K
NDT = 10

_MESH = plsc.VectorSubcoreMesh(core_axis_name="c", subcore_axis_name="s",
                               num_cores=NC, num_subcores=NS)


def _zero_acc(z_hbm, zidx_hbm, acc, zbuf, zv, s):
    pltpu.sync_copy(z_hbm, zbuf)
    pltpu.sync_copy(zidx_hbm.at[s], zv)

    def zbody(i, carry):
        pltpu.sync_copy(zbuf, acc.at[zv.at[i]])
        return carry

    lax.fori_loop(0, NZB, zbody, 0)


def _sc_body(h_hbm, src_hbm, dst_hbm, z_hbm, zidx_hbm,
             out_rows, acc, src_v, dst_v, zv, rows0, rows1, sem0, sem1):
    c = lax.axis_index("c")
    s = lax.axis_index("s")
    wid = s * NC + c
    _zero_acc(z_hbm, zidx_hbm, acc, rows0, zv, s)
    plsc.subcore_barrier()

    bufs = (rows0, rows1)
    sems = (sem0, sem1)

    def quarter(q, carry):
        pltpu.sync_copy(src_hbm.at[wid, q], src_v)
        pltpu.sync_copy(dst_hbm.at[wid, q], dst_v)
        pltpu.async_copy(h_hbm.at[src_v.at[0]], rows0, sem0)
        for j in range(QCHUNK):
            if j + 1 < QCHUNK:
                pltpu.async_copy(h_hbm.at[src_v.at[j + 1]],
                                 bufs[(j + 1) % 2], sems[(j + 1) % 2])
            pltpu.make_async_copy(h_hbm.at[src_v.at[j]], bufs[j % 2],
                                  sems[j % 2]).wait()
            pltpu.sync_copy(bufs[j % 2], acc.at[dst_v.at[j]], add=True)
        return carry

    lax.fori_loop(0, NQ, quarter, 0)
    plsc.subcore_barrier()

    @pl.when(s < NDT)
    def _drain():
        stripe = pl.ds(s * (N // NDT), N // NDT)
        pltpu.sync_copy(acc.at[stripe], out_rows.at[c, stripe])


_sc_agg = pl.kernel(
    _sc_body,
    out_type=jax.ShapeDtypeStruct((NC, N, D), jnp.float32),
    mesh=_MESH,
    scratch_types=[
        pltpu.VMEM_SHARED((NA, D), jnp.float32),
        pltpu.VMEM((QCHUNK, K), jnp.int32),
        pltpu.VMEM((QCHUNK, K), jnp.int32),
        pltpu.VMEM((NZB, K), jnp.int32),
        pltpu.VMEM((K, D), jnp.float32),
        pltpu.VMEM((K, D), jnp.float32),
        pltpu.SemaphoreType.DMA,
        pltpu.SemaphoreType.DMA,
    ],
)


NV = EPT // 16
HC = NA // 128


def _sc_cnt_body(dst_hbm, hidx_hbm, out_cnt, hist_sp, dst_v, hist, hidx_v):
    c = lax.axis_index("c")
    s = lax.axis_index("s")
    wid = s * NC + c
    zrow = jnp.zeros((16,), jnp.float32)

    def zb(i, carry):
        hist[pl.ds(i * 16, 16)] = zrow
        return carry

    lax.fori_loop(0, NV, zb, 0)
    pltpu.sync_copy(hist.at[pl.ds(0, NA // NS)],
                    hist_sp.at[pl.ds(s * (NA // NS), NA // NS)])
    pltpu.sync_copy(dst_hbm.at[wid], dst_v)
    pltpu.sync_copy(hidx_hbm, hidx_v)
    ones = jnp.ones((16,), jnp.float32)
    plsc.subcore_barrier()

    def hb(e, carry):
        v = dst_v[pl.ds(e * 16, 16)]
        plsc.addupdate_scatter(hist, [v], ones)
        return carry

    lax.fori_loop(0, NV, hb, 0)

    def rb(i, carry):
        pltpu.sync_copy(hist.at[pl.ds(i * 128, 128)],
                        hist_sp.at[hidx_v.at[i]], add=True)
        return carry

    lax.fori_loop(0, HC, rb, 0)
    plsc.subcore_barrier()

    @pl.when(s < 8)
    def _drain():
        stripe = pl.ds(s * (NA // 8), NA // 8)
        pltpu.sync_copy(hist_sp.at[stripe], out_cnt.at[c, stripe])


_sc_cnt = pl.kernel(
    _sc_cnt_body,
    out_type=jax.ShapeDtypeStruct((NC, NA), jnp.float32),
    mesh=_MESH,
    compiler_params=pltpu.CompilerParams(needs_layout_passes=False),
    scratch_types=[
        pltpu.VMEM_SHARED((NA,), jnp.float32),
        pltpu.VMEM((EPT,), jnp.int32),
        pltpu.VMEM((NA,), jnp.float32),
        pltpu.VMEM((HC, 128), jnp.int32),
    ],
)


BN = 1000


def _tc_body(p0, p1, c0, c1, h, wl, wr, b, o, *, act):
    cnt = c0[:, :] + c1[:, :]
    inv = 1.0 / jnp.maximum(cnt, 1.0)
    agg = (p0[:, :] + p1[:, :]) * inv
    y = (jnp.dot(agg, wl[:, :], preferred_element_type=jnp.float32)
         + jnp.dot(h[:, :], wr[:, :], preferred_element_type=jnp.float32)
         + b[:, :])
    o[:, :] = jnp.tanh(y) if act else y


def _tc_layer(parts, cnts, h, Wl, Wr, b, act):
    return pl.pallas_call(
        functools.partial(_tc_body, act=act),
        grid=(N // BN,),
        in_specs=[
            pl.BlockSpec((BN, D), lambda i: (i, 0)),
            pl.BlockSpec((BN, D), lambda i: (i, 0)),
            pl.BlockSpec((BN, 1), lambda i: (i, 0)),
            pl.BlockSpec((BN, 1), lambda i: (i, 0)),
            pl.BlockSpec((BN, D), lambda i: (i, 0)),
            pl.BlockSpec((D, D), lambda i: (0, 0)),
            pl.BlockSpec((D, D), lambda i: (0, 0)),
            pl.BlockSpec((1, D), lambda i: (0, 0)),
        ],
        out_specs=pl.BlockSpec((BN, D), lambda i: (i, 0)),
        out_shape=jax.ShapeDtypeStruct((N, D), jnp.float32),
    )(parts[0], parts[1], cnts[0], cnts[1], h, Wl, Wr, b.reshape(1, D))


def kernel(x, edge_index, Wl0, Wr0, b0, Wl1, Wr1, b1, Wl2, Wr2, b2):
    pad = EPAD - E
    padi = jnp.arange(pad, dtype=jnp.int32)
    srcf = jnp.concatenate([edge_index[0].astype(jnp.int32), padi % N])
    dstf = jnp.concatenate([edge_index[1].astype(jnp.int32),
                            N + padi % (NA - N)])
    src = srcf.reshape(NW, NQ, QCHUNK, K)
    dst = dstf.reshape(NW, NQ, QCHUNK, K)
    dstc = dstf.reshape(NW, EPT)
    z = jnp.zeros((K, D), jnp.float32)
    zidx = jnp.arange(NA, dtype=jnp.int32).reshape(NS, NZB, K)
    hidx = jnp.arange(NA, dtype=jnp.int32).reshape(HC, 128)

    cnts = _sc_cnt(dstc, hidx)
    cnts = cnts[:, :N].reshape(NC, N, 1)
    parts = _sc_agg(x, src, dst, z, zidx)
    h = _tc_layer(parts, cnts, x, Wl0, Wr0, b0, act=True)
    parts = _sc_agg(h, src, dst, z, zidx)
    h = _tc_layer(parts, cnts, h, Wl1, Wr1, b1, act=True)
    parts = _sc_agg(h, src, dst, z, zidx)
    return _tc_layer(parts, cnts, h, Wl2, Wr2, b2, act=False)
